# Initial kernel scaffold; baseline (speedup 1.0000x reference)
import jax
import jax.numpy as jnp
from jax.experimental import pallas as pl


def kernel(X, emb, W1, b1, W2, b2):
    raise NotImplementedError("write your pallas kernel here")


# speedup vs baseline: 128.7173x; 128.7173x over previous
import functools

import jax
import jax.numpy as jnp
from jax.experimental import pallas as pl


def _table_kernel(emb_ref, w1t_ref, out_ref):
    er = jax.nn.relu(emb_ref[...])
    out_ref[...] = jax.lax.dot_general(
        er, w1t_ref[...], (((1,), (0,)), ((), ())),
        preferred_element_type=jnp.float32)


def _mlp_kernel(tf_ref, x_ref, b1_ref, w2_ref, b2_ref, out_ref, *,
                n_vocab, l_seq):
    xb = x_ref[...]
    h = None
    for v in range(n_vocab):
        m = (xb == v).astype(jnp.float32)
        t_v = tf_ref[v * l_seq:(v + 1) * l_seq, :]
        d = jnp.dot(m, t_v, preferred_element_type=jnp.float32)
        h = d if h is None else h + d
    h = jax.nn.relu(h + b1_ref[...])
    out_ref[...] = jnp.dot(
        h, w2_ref[...], preferred_element_type=jnp.float32) + b2_ref[...]


def kernel(X, emb, W1, b1, W2, b2):
    B, L = X.shape
    V, D0 = emb.shape
    D1 = W1.shape[1]
    D2 = W2.shape[1]
    X = X.astype(jnp.int32)
    W1t = W1.reshape(L, D0, D1).transpose(1, 0, 2).reshape(D0, L * D1)

    t_flat = pl.pallas_call(
        _table_kernel,
        out_shape=jax.ShapeDtypeStruct((V, L * D1), jnp.float32),
    )(emb, W1t)
    tf = t_flat.reshape(V * L, D1)

    BB = 1024
    out = pl.pallas_call(
        functools.partial(_mlp_kernel, n_vocab=V, l_seq=L),
        grid=(B // BB,),
        in_specs=[
            pl.BlockSpec((V * L, D1), lambda i: (0, 0)),
            pl.BlockSpec((BB, L), lambda i: (i, 0)),
            pl.BlockSpec((1, D1), lambda i: (0, 0)),
            pl.BlockSpec((D1, D2), lambda i: (0, 0)),
            pl.BlockSpec((1, D2), lambda i: (0, 0)),
        ],
        out_specs=pl.BlockSpec((BB, D2), lambda i: (i, 0)),
        out_shape=jax.ShapeDtypeStruct((B, D2), jnp.float32),
    )(tf, X, b1.reshape(1, D1), W2, b2.reshape(1, D2))
    return out


# speedup vs baseline: 129.3711x; 1.0051x over previous
import functools

import jax
import jax.numpy as jnp
from jax.experimental import pallas as pl


def _table_kernel(emb_ref, w1t_ref, out_ref):
    er = jax.nn.relu(emb_ref[...])
    out_ref[...] = jax.lax.dot_general(
        er, w1t_ref[...], (((1,), (0,)), ((), ())),
        preferred_element_type=jnp.float32).astype(jnp.bfloat16)


def _mlp_kernel(tf_ref, x_ref, b1_ref, w2_ref, b2_ref, out_ref, *,
                n_vocab, l_seq):
    xb = x_ref[...]
    h = None
    for v in range(n_vocab):
        m = (xb == v).astype(jnp.bfloat16)
        t_v = tf_ref[v * l_seq:(v + 1) * l_seq, :]
        d = jnp.dot(m, t_v, preferred_element_type=jnp.float32)
        h = d if h is None else h + d
    h = jax.nn.relu(h + b1_ref[...]).astype(jnp.bfloat16)
    out_ref[...] = jnp.dot(
        h, w2_ref[...], preferred_element_type=jnp.float32) + b2_ref[...]


def kernel(X, emb, W1, b1, W2, b2):
    B, L = X.shape
    V, D0 = emb.shape
    D1 = W1.shape[1]
    D2 = W2.shape[1]
    X = X.astype(jnp.int32)
    W1t = W1.reshape(L, D0, D1).transpose(1, 0, 2).reshape(D0, L * D1)

    t_flat = pl.pallas_call(
        _table_kernel,
        out_shape=jax.ShapeDtypeStruct((V, L * D1), jnp.bfloat16),
    )(emb, W1t)
    tf = t_flat.reshape(V * L, D1)

    BB = 1024
    out = pl.pallas_call(
        functools.partial(_mlp_kernel, n_vocab=V, l_seq=L),
        grid=(B // BB,),
        in_specs=[
            pl.BlockSpec((V * L, D1), lambda i: (0, 0)),
            pl.BlockSpec((BB, L), lambda i: (i, 0)),
            pl.BlockSpec((1, D1), lambda i: (0, 0)),
            pl.BlockSpec((D1, D2), lambda i: (0, 0)),
            pl.BlockSpec((1, D2), lambda i: (0, 0)),
        ],
        out_specs=pl.BlockSpec((BB, D2), lambda i: (i, 0)),
        out_shape=jax.ShapeDtypeStruct((B, D2), jnp.float32),
    )(tf, X, b1.reshape(1, D1), W2.astype(jnp.bfloat16), b2.reshape(1, D2))
    return out


# speedup vs baseline: 132.1308x; 1.0213x over previous
import functools

import jax
import jax.numpy as jnp
from jax.experimental import pallas as pl


def _table_kernel(emb_ref, w1t_ref, out_ref):
    er = jax.nn.relu(emb_ref[...])
    out_ref[...] = jax.lax.dot_general(
        er, w1t_ref[...], (((1,), (0,)), ((), ())),
        preferred_element_type=jnp.float32).astype(jnp.bfloat16)


def _mlp_kernel(tf_ref, x_ref, b1_ref, w2_ref, b2_ref, out_ref, *,
                n_vocab, l_seq):
    xb = x_ref[...]
    h = None
    for v in range(n_vocab):
        m = (xb == v).astype(jnp.bfloat16)
        t_v = tf_ref[v * l_seq:(v + 1) * l_seq, :]
        d = jnp.dot(m, t_v, preferred_element_type=jnp.float32)
        h = d if h is None else h + d
    h = jax.nn.relu(h + b1_ref[...]).astype(jnp.bfloat16)
    out_ref[...] = jnp.dot(
        h, w2_ref[...], preferred_element_type=jnp.float32) + b2_ref[...]


def kernel(X, emb, W1, b1, W2, b2):
    B, L = X.shape
    V, D0 = emb.shape
    D1 = W1.shape[1]
    D2 = W2.shape[1]
    X = X.astype(jnp.int32)
    W1t = W1.reshape(L, D0, D1).transpose(1, 0, 2).reshape(D0, L * D1)

    t_flat = pl.pallas_call(
        _table_kernel,
        out_shape=jax.ShapeDtypeStruct((V, L * D1), jnp.bfloat16),
    )(emb, W1t)
    tf = t_flat.reshape(V * L, D1)

    BB = 2048
    out = pl.pallas_call(
        functools.partial(_mlp_kernel, n_vocab=V, l_seq=L),
        grid=(B // BB,),
        in_specs=[
            pl.BlockSpec((V * L, D1), lambda i: (0, 0)),
            pl.BlockSpec((BB, L), lambda i: (i, 0)),
            pl.BlockSpec((1, D1), lambda i: (0, 0)),
            pl.BlockSpec((D1, D2), lambda i: (0, 0)),
            pl.BlockSpec((1, D2), lambda i: (0, 0)),
        ],
        out_specs=pl.BlockSpec((BB, D2), lambda i: (i, 0)),
        out_shape=jax.ShapeDtypeStruct((B, D2), jnp.float32),
    )(tf, X, b1.reshape(1, D1), W2.astype(jnp.bfloat16), b2.reshape(1, D2))
    return out
